# baseline (speedup 1.0000x reference)
import jax
import jax.numpy as jnp
from jax import lax
from jax.experimental import pallas as pl
from jax.experimental.pallas import tpu as pltpu
from jax.experimental.pallas import tpu_sc as plsc

N_TRIPLETS = 320000
D = 128
DP = D // 2
NC = 2
NS = 16
NW = NC * NS
PW = N_TRIPLETS // NW
C = 80
K = PW // C
LANES = 16


def _sc_body(emb_hbm, wrel_hbm, trip_hbm, out_hbm,
             trip_v, sidx_v, ridx_v, oidx_v,
             sbuf0, rbuf0, obuf0, sbuf1, rbuf1, obuf1, scores_v,
             sem0, sem1):
    wid = lax.axis_index("s") * NC + lax.axis_index("c")
    base = wid * PW

    pltpu.sync_copy(trip_hbm.at[pl.ds(base * 3, PW * 3)], trip_v)
    iota16 = lax.iota(jnp.int32, LANES)
    iota3 = iota16 * 3

    def split_body(j, carry):
        flat = j * (LANES * 3) + iota3
        sidx_v[pl.ds(j * LANES, LANES)] = plsc.load_gather(trip_v, [flat])
        ridx_v[pl.ds(j * LANES, LANES)] = plsc.load_gather(trip_v, [flat + 1])
        oidx_v[pl.ds(j * LANES, LANES)] = plsc.load_gather(trip_v, [flat + 2])
        return carry

    lax.fori_loop(0, PW // LANES, split_body, 0, unroll=4)

    bufs = ((sbuf0, rbuf0, obuf0, sem0), (sbuf1, rbuf1, obuf1, sem1))

    def issue(c, slot):
        sb, rb, ob, sem = bufs[slot]
        off = c * C
        pltpu.async_copy(emb_hbm.at[sidx_v.at[pl.ds(off, C)]], sb, sem)
        pltpu.async_copy(wrel_hbm.at[ridx_v.at[pl.ds(off, C)]], rb, sem)
        pltpu.async_copy(emb_hbm.at[oidx_v.at[pl.ds(off, C)]], ob, sem)

    def wait(c, slot):
        sb, rb, ob, sem = bufs[slot]
        off = c * C
        pltpu.make_async_copy(emb_hbm.at[sidx_v.at[pl.ds(off, C)]], sb, sem).wait()
        pltpu.make_async_copy(wrel_hbm.at[ridx_v.at[pl.ds(off, C)]], rb, sem).wait()
        pltpu.make_async_copy(emb_hbm.at[oidx_v.at[pl.ds(off, C)]], ob, sem).wait()

    def compute(c, slot):
        sb, rb, ob, _ = bufs[slot]
        off = c * C
        for t in range(C // LANES):
            rows = t * LANES + iota16

            def dbody(dd, acc):
                cols = (iota16 + dd) & (DP - 1)
                sv = plsc.bitcast(plsc.load_gather(sb, [rows, cols]),
                                  jnp.bfloat16)
                rv = plsc.bitcast(plsc.load_gather(rb, [rows, cols]),
                                  jnp.bfloat16)
                ov = plsc.bitcast(plsc.load_gather(ob, [rows, cols]),
                                  jnp.bfloat16)
                pa, pb = plsc.unpack(sv * rv * ov,
                                     format=plsc.PackFormat.INTERLEAVED)
                return acc + pa + pb

            acc = lax.fori_loop(0, DP, dbody, jnp.zeros((LANES,), jnp.float32),
                                unroll=8)
            scores_v[pl.ds(off + t * LANES, LANES)] = acc

    issue(0, 0)

    def pair_body(p, carry):
        c0 = 2 * p
        issue(c0 + 1, 1)
        wait(c0, 0)
        compute(c0, 0)
        issue(c0 + 2, 0)
        wait(c0 + 1, 1)
        compute(c0 + 1, 1)
        return carry

    lax.fori_loop(0, (K - 1) // 2, pair_body, 0)
    wait(K - 1, 0)
    compute(K - 1, 0)

    pltpu.sync_copy(scores_v, out_hbm.at[pl.ds(base, PW)])


def _pack_table(table):
    t16 = lax.bitcast_convert_type(table.astype(jnp.bfloat16), jnp.uint16)
    lo = t16[:, 0::2].astype(jnp.uint32)
    hi = t16[:, 1::2].astype(jnp.uint32)
    return lax.bitcast_convert_type(lo | (hi << 16), jnp.int32)


@jax.jit
def kernel(embedding, triplets, w_relation):
    trip = triplets.astype(jnp.int32).reshape(-1)
    emb_p = _pack_table(embedding)
    wrel_p = _pack_table(w_relation)

    mesh = plsc.VectorSubcoreMesh(core_axis_name="c", subcore_axis_name="s")
    row_buf = pltpu.VMEM((C, DP), jnp.int32)
    k = pl.kernel(
        _sc_body,
        out_type=jax.ShapeDtypeStruct((N_TRIPLETS,), jnp.float32),
        mesh=mesh,
        compiler_params=pltpu.CompilerParams(needs_layout_passes=False,
                                             use_tc_tiling_on_sc=False),
        scratch_types=[
            pltpu.VMEM((PW * 3,), jnp.int32),
            pltpu.VMEM((PW,), jnp.int32),
            pltpu.VMEM((PW,), jnp.int32),
            pltpu.VMEM((PW,), jnp.int32),
            row_buf, row_buf, row_buf,
            row_buf, row_buf, row_buf,
            pltpu.VMEM((PW,), jnp.float32),
            pltpu.SemaphoreType.DMA,
            pltpu.SemaphoreType.DMA,
        ],
    )
    return k(emb_p, wrel_p, trip)


# speedup vs baseline: 3.9390x; 3.9390x over previous
import jax
import jax.numpy as jnp
from jax import lax
from jax.experimental import pallas as pl
from jax.experimental.pallas import tpu as pltpu
from jax.experimental.pallas import tpu_sc as plsc

N_TRIPLETS = 320000
D = 128
DP = D // 2
NC = 2
NS = 16
NW = NC * NS
PW = N_TRIPLETS // NW
C = 80
K = PW // C
LANES = 16


def _sc_body(emb_hbm, wrel_hbm, sidx_hbm, ridx_hbm, oidx_hbm, out_hbm,
             sidx_v, ridx_v, oidx_v,
             sbuf0, rbuf0, obuf0, sbuf1, rbuf1, obuf1, scores_v,
             sem0, sem1):
    wid = lax.axis_index("s") * NC + lax.axis_index("c")
    base = wid * PW

    pltpu.sync_copy(sidx_hbm.at[pl.ds(base, PW)], sidx_v)
    pltpu.sync_copy(ridx_hbm.at[pl.ds(base, PW)], ridx_v)
    pltpu.sync_copy(oidx_hbm.at[pl.ds(base, PW)], oidx_v)

    bufs = ((sbuf0, rbuf0, obuf0, sem0), (sbuf1, rbuf1, obuf1, sem1))

    def issue(c, slot):
        sb, rb, ob, sem = bufs[slot]
        off = c * C
        pltpu.async_copy(emb_hbm.at[sidx_v.at[pl.ds(off, C)]], sb, sem)
        pltpu.async_copy(wrel_hbm.at[ridx_v.at[pl.ds(off, C)]], rb, sem)
        pltpu.async_copy(emb_hbm.at[oidx_v.at[pl.ds(off, C)]], ob, sem)

    def wait(c, slot):
        sb, rb, ob, sem = bufs[slot]
        off = c * C
        pltpu.make_async_copy(emb_hbm.at[sidx_v.at[pl.ds(off, C)]], sb, sem).wait()
        pltpu.make_async_copy(wrel_hbm.at[ridx_v.at[pl.ds(off, C)]], rb, sem).wait()
        pltpu.make_async_copy(emb_hbm.at[oidx_v.at[pl.ds(off, C)]], ob, sem).wait()

    iota16 = lax.iota(jnp.int32, LANES)

    def compute(c, slot):
        sb, rb, ob, _ = bufs[slot]
        off = c * C
        for t in range(C // LANES):
            rows = t * LANES + iota16

            def dbody(dd, acc):
                cols = (iota16 + dd) & (DP - 1)
                sv = plsc.bitcast(plsc.load_gather(sb, [rows, cols]),
                                  jnp.bfloat16)
                rv = plsc.bitcast(plsc.load_gather(rb, [rows, cols]),
                                  jnp.bfloat16)
                ov = plsc.bitcast(plsc.load_gather(ob, [rows, cols]),
                                  jnp.bfloat16)
                pa, pb = plsc.unpack(sv * rv * ov,
                                     format=plsc.PackFormat.INTERLEAVED)
                return acc + pa + pb

            acc = lax.fori_loop(0, DP, dbody, jnp.zeros((LANES,), jnp.float32),
                                unroll=8)
            scores_v[pl.ds(off + t * LANES, LANES)] = acc

    issue(0, 0)

    def pair_body(p, carry):
        c0 = 2 * p
        issue(c0 + 1, 1)
        wait(c0, 0)
        compute(c0, 0)
        issue(c0 + 2, 0)
        wait(c0 + 1, 1)
        compute(c0 + 1, 1)
        return carry

    lax.fori_loop(0, (K - 1) // 2, pair_body, 0)
    wait(K - 1, 0)
    compute(K - 1, 0)

    pltpu.sync_copy(scores_v, out_hbm.at[pl.ds(base, PW)])


def _pack_table(table):
    t16 = lax.bitcast_convert_type(table.astype(jnp.bfloat16), jnp.uint16)
    lo = t16[:, :DP].astype(jnp.uint32)
    hi = t16[:, DP:].astype(jnp.uint32)
    return lax.bitcast_convert_type(lo | (hi << 16), jnp.int32)


@jax.jit
def kernel(embedding, triplets, w_relation):
    trip = triplets.astype(jnp.int32)
    s_idx = trip[:, 0]
    r_idx = trip[:, 1]
    o_idx = trip[:, 2]
    emb_p = _pack_table(embedding)
    wrel_p = _pack_table(w_relation)

    mesh = plsc.VectorSubcoreMesh(core_axis_name="c", subcore_axis_name="s")
    row_buf = pltpu.VMEM((C, DP), jnp.int32)
    k = pl.kernel(
        _sc_body,
        out_type=jax.ShapeDtypeStruct((N_TRIPLETS,), jnp.float32),
        mesh=mesh,
        compiler_params=pltpu.CompilerParams(needs_layout_passes=False,
                                             use_tc_tiling_on_sc=False),
        scratch_types=[
            pltpu.VMEM((PW,), jnp.int32),
            pltpu.VMEM((PW,), jnp.int32),
            pltpu.VMEM((PW,), jnp.int32),
            row_buf, row_buf, row_buf,
            row_buf, row_buf, row_buf,
            pltpu.VMEM((PW,), jnp.float32),
            pltpu.SemaphoreType.DMA,
            pltpu.SemaphoreType.DMA,
        ],
    )
    return k(emb_p, wrel_p, s_idx, r_idx, o_idx)


# speedup vs baseline: 4.2154x; 1.0702x over previous
import jax
import jax.numpy as jnp
from jax import lax
from jax.experimental import pallas as pl
from jax.experimental.pallas import tpu as pltpu
from jax.experimental.pallas import tpu_sc as plsc

N_TRIPLETS = 320000
D = 128
DP = D // 2
NC = 2
NS = 16
NW = NC * NS
PW = N_TRIPLETS // NW
C = 80
K = PW // C
LANES = 16


def _sc_body(emb_hbm, wrel_hbm, tripT_hbm, out_hbm,
             sidx_v, ridx_v, oidx_v,
             sbuf0, rbuf0, obuf0, sbuf1, rbuf1, obuf1, scores_v,
             sem0, sem1):
    wid = lax.axis_index("s") * NC + lax.axis_index("c")
    base = wid * PW

    pltpu.sync_copy(tripT_hbm.at[0, pl.ds(base, PW)], sidx_v)
    pltpu.sync_copy(tripT_hbm.at[1, pl.ds(base, PW)], ridx_v)
    pltpu.sync_copy(tripT_hbm.at[2, pl.ds(base, PW)], oidx_v)

    bufs = ((sbuf0, rbuf0, obuf0, sem0), (sbuf1, rbuf1, obuf1, sem1))

    def issue(c, slot):
        sb, rb, ob, sem = bufs[slot]
        off = c * C
        pltpu.async_copy(emb_hbm.at[sidx_v.at[pl.ds(off, C)]], sb, sem)
        pltpu.async_copy(wrel_hbm.at[ridx_v.at[pl.ds(off, C)]], rb, sem)
        pltpu.async_copy(emb_hbm.at[oidx_v.at[pl.ds(off, C)]], ob, sem)

    def wait(c, slot):
        sb, rb, ob, sem = bufs[slot]
        off = c * C
        pltpu.make_async_copy(emb_hbm.at[sidx_v.at[pl.ds(off, C)]], sb, sem).wait()
        pltpu.make_async_copy(wrel_hbm.at[ridx_v.at[pl.ds(off, C)]], rb, sem).wait()
        pltpu.make_async_copy(emb_hbm.at[oidx_v.at[pl.ds(off, C)]], ob, sem).wait()

    iota16 = lax.iota(jnp.int32, LANES)

    def compute(c, slot):
        sb, rb, ob, _ = bufs[slot]
        off = c * C
        for t in range(C // LANES):
            rows = t * LANES + iota16

            def dbody(dd, acc):
                cols = (iota16 + dd) & (DP - 1)
                sv = plsc.bitcast(plsc.load_gather(sb, [rows, cols]),
                                  jnp.bfloat16)
                rv = plsc.bitcast(plsc.load_gather(rb, [rows, cols]),
                                  jnp.bfloat16)
                ov = plsc.bitcast(plsc.load_gather(ob, [rows, cols]),
                                  jnp.bfloat16)
                pa, pb = plsc.unpack(sv * rv * ov,
                                     format=plsc.PackFormat.INTERLEAVED)
                return acc + pa + pb

            acc = lax.fori_loop(0, DP, dbody, jnp.zeros((LANES,), jnp.float32),
                                unroll=8)
            scores_v[pl.ds(off + t * LANES, LANES)] = acc

    issue(0, 0)

    def pair_body(p, carry):
        c0 = 2 * p
        issue(c0 + 1, 1)
        wait(c0, 0)
        compute(c0, 0)
        issue(c0 + 2, 0)
        wait(c0 + 1, 1)
        compute(c0 + 1, 1)
        return carry

    lax.fori_loop(0, (K - 1) // 2, pair_body, 0)
    wait(K - 1, 0)
    compute(K - 1, 0)

    pltpu.sync_copy(scores_v, out_hbm.at[pl.ds(base, PW)])


def _pack_table(table):
    t16 = lax.bitcast_convert_type(table.astype(jnp.bfloat16), jnp.uint16)
    lo = t16[:, :DP].astype(jnp.uint32)
    hi = t16[:, DP:].astype(jnp.uint32)
    return lax.bitcast_convert_type(lo | (hi << 16), jnp.int32)


@jax.jit
def kernel(embedding, triplets, w_relation):
    trip_t = triplets.astype(jnp.int32).T
    emb_p = _pack_table(embedding)
    wrel_p = _pack_table(w_relation)

    mesh = plsc.VectorSubcoreMesh(core_axis_name="c", subcore_axis_name="s")
    row_buf = pltpu.VMEM((C, DP), jnp.int32)
    k = pl.kernel(
        _sc_body,
        out_type=jax.ShapeDtypeStruct((N_TRIPLETS,), jnp.float32),
        mesh=mesh,
        compiler_params=pltpu.CompilerParams(needs_layout_passes=False,
                                             use_tc_tiling_on_sc=False),
        scratch_types=[
            pltpu.VMEM((PW,), jnp.int32),
            pltpu.VMEM((PW,), jnp.int32),
            pltpu.VMEM((PW,), jnp.int32),
            row_buf, row_buf, row_buf,
            row_buf, row_buf, row_buf,
            pltpu.VMEM((PW,), jnp.float32),
            pltpu.SemaphoreType.DMA,
            pltpu.SemaphoreType.DMA,
        ],
    )
    return k(emb_p, wrel_p, trip_t)
